# baseline (speedup 1.0000x reference)
import functools

import jax
import jax.numpy as jnp
from jax import lax
from jax.experimental import pallas as pl
from jax.experimental.pallas import tpu as pltpu
from jax.experimental.pallas import tpu_sc as plsc

NUM_ITEMS = 100000
EMBED_DIM = 64
BATCH = 16384
PAD_DIM = 128
TILE_ROWS = 8
N_TILES = NUM_ITEMS // TILE_ROWS

_NC = 2
_NS = 16
_NW = _NC * _NS
_B_PER_W = BATCH // _NW
_L = 16
_OCH = 256
_NOCH = _B_PER_W // _OCH
_NG = _OCH // _L

_mesh = plsc.VectorSubcoreMesh(core_axis_name="c", subcore_axis_name="s")


@functools.partial(
    pl.kernel,
    mesh=_mesh,
    out_type=jax.ShapeDtypeStruct((BATCH, PAD_DIM), jnp.float32),
    scratch_types=[
        pltpu.VMEM((_B_PER_W,), jnp.int32),
        pltpu.VMEM((2, _OCH, PAD_DIM), jnp.float32),
        pltpu.VMEM((_OCH * EMBED_DIM,), jnp.int32),
        pltpu.SemaphoreType.DMA,
        pltpu.SemaphoreType.DMA,
        pltpu.SemaphoreType.DMA,
    ],
    compiler_params=pltpu.CompilerParams(
        use_tc_tiling_on_sc=True, needs_layout_passes=False
    ),
)
def _gather_kernel(idx_hbm, table_hbm, out_hbm, idx_v, rows_v, dummy_v,
                   semA, semB, osem):
    wid = lax.axis_index("s") * _NC + lax.axis_index("c")
    base = wid * _B_PER_W
    pltpu.sync_copy(idx_hbm.at[pl.ds(base, _B_PER_W)], idx_v)
    sems = (semA, semB)

    def _fire_chunk(j):
        sem = sems[j % 2]
        buf = j % 2

        def _group(g, carry):
            iv = idx_v[pl.ds(g * _L, _L)]
            tv = lax.shift_right_logical(iv, 3)
            rv = lax.rem(iv, TILE_ROWS)
            for t in range(_L):
                pltpu.async_copy(
                    table_hbm.at[tv[t], rv[t]],
                    rows_v.at[buf, lax.rem(g, _NG) * _L + t,
                              pl.ds(0, EMBED_DIM)],
                    sem,
                )
            return carry

        lax.fori_loop(j * _NG, (j + 1) * _NG, _group, 0, unroll=False)

    def _drain_chunk(j):
        pltpu.make_async_copy(
            idx_hbm.at[pl.ds(0, _OCH * EMBED_DIM)], dummy_v, sems[j % 2]
        ).wait()

    _fire_chunk(0)
    for j in range(_NOCH):
        if j + 1 < _NOCH:
            if j >= 1:
                pltpu.make_async_copy(
                    rows_v.at[0], out_hbm.at[pl.ds(0, _OCH)], osem
                ).wait()
            _fire_chunk(j + 1)
        _drain_chunk(j)
        pltpu.async_copy(
            rows_v.at[j % 2],
            out_hbm.at[pl.ds(base + j * _OCH, _OCH)],
            osem,
        )
    for _ in range(2):
        pltpu.make_async_copy(
            rows_v.at[0], out_hbm.at[pl.ds(0, _OCH)], osem
        ).wait()


def kernel(item_indices, embedding_table):
    tiled = jnp.reshape(embedding_table, (N_TILES, TILE_ROWS, EMBED_DIM))
    padded = _gather_kernel(item_indices.astype(jnp.int32), tiled)
    return padded[:, :EMBED_DIM]
